# baseline (speedup 1.0000x reference)
import functools

import jax
import jax.numpy as jnp
from jax import lax
from jax.experimental import pallas as pl
from jax.experimental.pallas import tpu as pltpu
from jax.experimental.pallas import tpu_sc as plsc

_N = 100000
_L = 12800
_NPAD = 8 * _L
_K = 1000
_HID = 128
_OUT = 256
_NT = 32
_TS = _NPAD // _NT
_NCHK = _TS // 128
_SLOTS = 1024
_TRASH = 1016


def _select_kernel(lig_ref, xc_ref, pr_ref, rankc_ref):
    cx = jnp.sum(lig_ref[0:1, :]) * (1.0 / 32.0)
    cy = jnp.sum(lig_ref[1:2, :]) * (1.0 / 32.0)
    cz = jnp.sum(lig_ref[2:3, :]) * (1.0 / 32.0)
    dx = pr_ref[0:8, :] - cx
    dy = pr_ref[8:16, :] - cy
    dz = pr_ref[16:24, :] - cz
    dist = jnp.sqrt(dx * dx + dy * dy + dz * dz)
    chem = (xc_ref[0:8, :] * 0.3 + xc_ref[8:16, :] * 0.4
            + xc_ref[16:24, :] * 0.3)
    score = jnp.exp(dist * (-1.0 / 8.0)) * 0.7 + chem * 0.3
    idx = (lax.broadcasted_iota(jnp.int32, (8, _L), 0) * _L
           + lax.broadcasted_iota(jnp.int32, (8, _L), 1))
    score = jnp.where(idx < _N, score, -1.0)
    s = lax.bitcast_convert_type(score, jnp.int32)

    def tbody(i, t):
        cand = t | jnp.left_shift(jnp.int32(1), 29 - i)
        cnt = jnp.sum((s >= cand).astype(jnp.int32))
        return jnp.where(cnt >= _K, cand, t)

    thr = lax.fori_loop(0, 30, tbody, jnp.int32(0))

    cnt_ge = jnp.sum((s >= thr).astype(jnp.int32))
    eq = s == thr

    def tie_cut():
        need = _K - (cnt_ge - jnp.sum(eq.astype(jnp.int32)))

        def ibody(i, c):
            cand = c | jnp.left_shift(jnp.int32(1), 16 - i)
            cnt = jnp.sum((eq & (idx < cand)).astype(jnp.int32))
            return jnp.where(cnt < need, cand, c)

        return lax.fori_loop(0, 17, ibody, jnp.int32(0))

    icut = lax.cond(cnt_ge == _K, lambda: jnp.int32(_NPAD), tie_cut)

    seli = ((s > thr) | (eq & (idx <= icut))).astype(jnp.int32)

    c = seli
    sh = 1
    while sh < _L:
        c = c + jnp.concatenate(
            [jnp.zeros((8, sh), jnp.int32), c[:, :_L - sh]], axis=1)
        sh *= 2
    tot = c[:, _L - 1:_L]
    p = jnp.concatenate([jnp.zeros((1, 1), jnp.int32), tot[:-1, :]], axis=0)
    for sh in (1, 2, 4):
        p = p + jnp.concatenate(
            [jnp.zeros((sh, 1), jnp.int32), p[:-sh, :]], axis=0)
    rank = c - seli + p
    rankc_ref[...] = jnp.where(seli > 0, rank, _TRASH)


@jax.jit
def _select(lig, xc, pr):
    return pl.pallas_call(
        _select_kernel,
        out_shape=jax.ShapeDtypeStruct((8, _L), jnp.int32),
    )(lig, xc, pr)


def _scatter_body(rankc_hbm, idxs_hbm, list_hbm, rank_v, vals_v, sem):
    wid = lax.axis_index("s") * 2 + lax.axis_index("c")
    pltpu.sync_copy(rankc_hbm.at[wid], rank_v)
    pltpu.sync_copy(idxs_hbm.at[wid], vals_v)
    copies = []
    for k in range(_NCHK):
        copies.append(
            pltpu.async_copy(vals_v.at[k], list_hbm.at[rank_v.at[k]], sem))
    for cp in copies:
        cp.wait()


@jax.jit
def _sc_scatter(rankc, idxs):
    mesh = plsc.VectorSubcoreMesh(core_axis_name="c", subcore_axis_name="s")
    kfn = functools.partial(
        pl.kernel,
        out_type=jax.ShapeDtypeStruct((_SLOTS,), jnp.int32),
        mesh=mesh,
        scratch_types=[
            pltpu.VMEM((_NCHK, 128), jnp.int32),
            pltpu.VMEM((_NCHK, 128), jnp.int32),
            pltpu.SemaphoreType.DMA,
        ],
    )(_scatter_body)
    return kfn(rankc, idxs)


def _gather_body(list_hbm, x_hbm, out_hbm, idx_v, rows_v, sem):
    wid = lax.axis_index("s") * 2 + lax.axis_index("c")
    base = wid * (_SLOTS // _NT)
    pltpu.sync_copy(list_hbm.at[pl.ds(base, 32)], idx_v)
    lanes = lax.iota(jnp.int32, 16)
    for cc in range(2):
        v = idx_v[pl.ds(cc * 16, 16)]
        slot = base + cc * 16 + lanes
        idx_v[pl.ds(cc * 16, 16)] = jnp.where(slot < _K, v, 0)
    pltpu.async_copy(x_hbm.at[idx_v], rows_v, sem).wait()
    pltpu.sync_copy(rows_v, out_hbm.at[pl.ds(base, 32)])


@jax.jit
def _sc_gather(lst, x):
    mesh = plsc.VectorSubcoreMesh(core_axis_name="c", subcore_axis_name="s")
    kfn = functools.partial(
        pl.kernel,
        out_type=jax.ShapeDtypeStruct((_SLOTS, 8), jnp.float32),
        mesh=mesh,
        compiler_params=pltpu.CompilerParams(use_tc_tiling_on_sc=False),
        scratch_types=[
            pltpu.VMEM((32,), jnp.int32),
            pltpu.VMEM((32, 8), jnp.float32),
            pltpu.SemaphoreType.DMA,
        ],
    )(_gather_body)
    return kfn(lst, x)


def _embed_kernel(cr_ref, wet_ref, be_ref, wo_ref, bo_ref, out_ref):
    valid = lax.broadcasted_iota(jnp.int32, (_SLOTS, 1), 0) < _K
    z = (jnp.dot(cr_ref[...], wet_ref[...],
                 preferred_element_type=jnp.float32) + be_ref[...])
    z = jnp.where(valid, z, 0.0)
    h = z / (1.0 + jnp.exp(-z))
    pooled = jnp.sum(h, axis=0, keepdims=True) * (1.0 / _K)
    out_ref[...] = jnp.dot(pooled, wo_ref[...],
                           preferred_element_type=jnp.float32) + bo_ref[...]


@jax.jit
def _embed(cr, wet, be, wo, bo):
    return pl.pallas_call(
        _embed_kernel,
        out_shape=jax.ShapeDtypeStruct((1, _OUT), jnp.float32),
    )(cr, wet, be, wo, bo)


def kernel(pocket_x, pocket_pos, ligand_pos, W_embed, b_embed, W_out, b_out):
    pad = _NPAD - _N
    pr = jnp.concatenate(
        [jnp.pad(pocket_pos[:, j], (0, pad)).reshape(8, _L)
         for j in range(3)], axis=0)
    xc = jnp.concatenate(
        [jnp.pad(pocket_x[:, j], (0, pad)).reshape(8, _L)
         for j in (2, 3, 5)], axis=0)
    rankc = _select(ligand_pos.T, xc, pr)
    idxs = jnp.arange(_NPAD, dtype=jnp.int32).reshape(_NT, _NCHK, 128)
    lst = _sc_scatter(rankc.reshape(_NT, _NCHK, 128), idxs)
    compact = _sc_gather(lst, pocket_x)
    out = _embed(compact, W_embed, b_embed.reshape(1, _HID),
                 W_out, b_out.reshape(1, _OUT))
    return out.reshape(_OUT)


# speedup vs baseline: 284.7209x; 284.7209x over previous
import jax
import jax.numpy as jnp
from jax.experimental import pallas as pl
from jax.experimental.pallas import tpu as pltpu

_N = 100000
_L = 12544
_NPAD = 8 * _L
_K = 1000
_HID = 128
_OUT = 256
_CH = 6272
_NCH = 16


def _pocket_kernel(lig_ref, xc_ref, pr_ref, xt_ref, wet_ref, be_ref, wo_ref,
                   bo_ref, out_ref, mask_ref):
    cx = jnp.sum(lig_ref[0:1, :]) * (1.0 / 32.0)
    cy = jnp.sum(lig_ref[1:2, :]) * (1.0 / 32.0)
    cz = jnp.sum(lig_ref[2:3, :]) * (1.0 / 32.0)
    dx = pr_ref[0:8, :] - cx
    dy = pr_ref[8:16, :] - cy
    dz = pr_ref[16:24, :] - cz
    dist = jnp.sqrt(dx * dx + dy * dy + dz * dz)
    chem = (xc_ref[0:8, :] * 0.3 + xc_ref[8:16, :] * 0.4
            + xc_ref[16:24, :] * 0.3)
    score = jnp.exp(dist * (-1.0 / 8.0)) * 0.7 + chem * 0.3
    idx = (jax.lax.broadcasted_iota(jnp.int32, (8, _L), 0) * _L
           + jax.lax.broadcasted_iota(jnp.int32, (8, _L), 1))
    score = jnp.where(idx < _N, score, -1.0)
    s = jax.lax.bitcast_convert_type(score, jnp.int32)

    def tbody(i, t):
        sh = 27 - 3 * i
        m_best = jnp.int32(0)
        for m in range(1, 8):
            cand = t | jnp.left_shift(jnp.int32(m), sh)
            cnt = jnp.sum((s >= cand).astype(jnp.int32))
            m_best = m_best + (cnt >= _K).astype(jnp.int32)
        return t | jnp.left_shift(m_best, sh)

    thr = jax.lax.fori_loop(0, 10, tbody, jnp.int32(0))

    cnt_ge = jnp.sum((s >= thr).astype(jnp.int32))
    eq = s == thr

    def tie_cut():
        need = _K - (cnt_ge - jnp.sum(eq.astype(jnp.int32)))

        def ibody(i, c):
            cand = c | jnp.left_shift(jnp.int32(1), 16 - i)
            cnt = jnp.sum((eq & (idx < cand)).astype(jnp.int32))
            return jnp.where(cnt < need, cand, c)

        return jax.lax.fori_loop(0, 17, ibody, jnp.int32(0))

    icut = jax.lax.cond(cnt_ge == _K, lambda: jnp.int32(_NPAD), tie_cut)

    sel = (s > thr) | (eq & (idx <= icut))
    mask2 = sel.astype(jnp.float32)
    for r in range(8):
        mask_ref[0:1, r * _L:(r + 1) * _L] = mask2[r:r + 1, :]

    wet = wet_ref[...]
    be = be_ref[...]

    def ebody(i, acc):
        off = pl.multiple_of(i * _CH, 128)
        xs = xt_ref[:, pl.ds(off, _CH)]
        mk = mask_ref[:, pl.ds(off, _CH)]
        z = jnp.dot(wet, xs, preferred_element_type=jnp.float32) + be
        z = z * mk
        h = z / (1.0 + jnp.exp(-z))
        return acc + jnp.sum(h, axis=1, keepdims=True)

    acc = jax.lax.fori_loop(0, _NCH, ebody,
                            jnp.zeros((_HID, 1), jnp.float32))
    pooled = acc * (1.0 / _K)
    out = jnp.sum(pooled * wo_ref[...], axis=0, keepdims=True) + bo_ref[...]
    out_ref[...] = out


@jax.jit
def _run(lig, xc, pr, xt, wet, be, wo, bo):
    return pl.pallas_call(
        _pocket_kernel,
        out_shape=jax.ShapeDtypeStruct((1, _OUT), jnp.float32),
        scratch_shapes=[pltpu.VMEM((1, _NPAD), jnp.float32)],
    )(lig, xc, pr, xt, wet, be, wo, bo)


def kernel(pocket_x, pocket_pos, ligand_pos, W_embed, b_embed, W_out, b_out):
    pad = _NPAD - _N
    xt = jnp.pad(pocket_x.T, ((0, 0), (0, pad)))
    pr = jnp.concatenate(
        [jnp.pad(pocket_pos[:, j], (0, pad)).reshape(8, _L)
         for j in range(3)], axis=0)
    xc = jnp.concatenate(
        [jnp.pad(pocket_x[:, j], (0, pad)).reshape(8, _L)
         for j in (2, 3, 5)], axis=0)
    lig = ligand_pos.T
    wet = W_embed.T
    be = b_embed.reshape(_HID, 1)
    bo = b_out.reshape(1, _OUT)
    out = _run(lig, xc, pr, xt, wet, be, W_out, bo)
    return out.reshape(_OUT)
